# baseline (speedup 1.0000x reference)
import dataclasses
import functools

import jax
import jax.numpy as jnp
from jax import lax
from jax.experimental import pallas as pl
from jax.experimental.pallas import tpu as pltpu
from jax.experimental.pallas import tpu_sc as plsc

H, W = 480, 640
PLANE = H * W
LANES = 16
LANE_TILE = 128
WTILES = W // LANE_TILE
NUM_WORKERS = 32
NUM_CORES = 2


def _build_gather(B, N):
    ntiles_b = (N + LANE_TILE - 1) // LANE_TILE
    t_total = B * ntiles_b
    e_total = B * N
    n_pad = ntiles_b * LANE_TILE
    slots = t_total * 2 * LANE_TILE
    pad_gap = n_pad - N

    tq, rem = divmod(t_total, NUM_WORKERS)
    CT = 49
    NCH = -(-(tq + 1) // CT)
    SRC_WORDS = CT * LANE_TILE
    OUT_WORDS = CT * 2 * LANE_TILE

    mesh = plsc.VectorSubcoreMesh(core_axis_name="c", subcore_axis_name="s")
    cp = pltpu.CompilerParams()
    if "needs_layout_passes" in pltpu.CompilerParams.__dataclass_fields__:
        cp = dataclasses.replace(cp, needs_layout_passes=False)

    @functools.partial(
        pl.kernel,
        out_type=jax.ShapeDtypeStruct((slots,), jnp.float32),
        mesh=mesh,
        compiler_params=cp,
        scratch_types=[
            pltpu.VMEM((SRC_WORDS + pad_gap,), jnp.int32),
            pltpu.VMEM((SRC_WORDS + pad_gap,), jnp.int32),
            pltpu.VMEM((OUT_WORDS,), jnp.int32),
            pltpu.VMEM((OUT_WORDS,), jnp.float32),
            pltpu.SemaphoreType.DMA,
        ],
    )
    def gather_kernel(flow_hbm, y_hbm, x_hbm, out_hbm, ybuf, xbuf, idxbuf,
                      obuf, sem):
        cid = lax.axis_index("c")
        sid = lax.axis_index("s")
        wid = sid * NUM_CORES + cid
        t0 = wid * tq + jnp.minimum(wid, rem)
        t1 = t0 + tq + jnp.where(wid < rem, 1, 0)

        zv = jnp.zeros((LANES,), jnp.int32)

        @pl.loop(0, pad_gap, step=LANES)
        def _z(j):
            ybuf[pl.ds(SRC_WORDS + j, LANES)] = zv
            xbuf[pl.ds(SRC_WORDS + j, LANES)] = zv

        @pl.loop(0, NCH)
        def _chunk(ci):
            S = jnp.minimum(t0 + ci * CT, t1 - CT)
            bS = S // ntiles_b
            eS = S * LANE_TILE - pad_gap * bS
            dma_base = jnp.minimum(eS, e_total - SRC_WORDS)
            comp = eS - dma_base
            pltpu.sync_copy(
                y_hbm.at[pl.ds(dma_base, SRC_WORDS)],
                ybuf.at[pl.ds(0, SRC_WORDS)],
            )
            pltpu.sync_copy(
                x_hbm.at[pl.ds(dma_base, SRC_WORDS)],
                xbuf.at[pl.ds(0, SRC_WORDS)],
            )

            @pl.loop(0, CT)
            def _tile(t):
                tt = S + t
                b = tt // ntiles_b
                srcoff = t * LANE_TILE - pad_gap * (b - bS) + comp
                toff = t * (2 * LANE_TILE)
                plane0 = b * (2 * PLANE)

                @pl.loop(0, LANE_TILE, step=LANES)
                def _vec(j):
                    yv = ybuf[pl.ds(srcoff + j, LANES)]
                    xv = xbuf[pl.ds(srcoff + j, LANES)]
                    pidx = (
                        ((yv >> 3) * WTILES + (xv >> 7)) * 1024
                        + ((yv & 7) << 7)
                        + (xv & 127)
                    )
                    idxbuf[pl.ds(toff + j, LANES)] = pidx + (plane0 + PLANE)
                    idxbuf[pl.ds(toff + LANE_TILE + j, LANES)] = pidx + plane0

            pltpu.async_copy(flow_hbm.at[idxbuf], obuf, sem).wait()
            pltpu.sync_copy(
                obuf, out_hbm.at[pl.ds(S * (2 * LANE_TILE), OUT_WORDS)]
            )

    return gather_kernel, ntiles_b, n_pad, slots


def kernel(flow, event_list, pol_mask, event_mask):
    B, _, h, w = flow.shape
    N = event_list.shape[1]
    gk, ntiles_b, n_pad, slots = _build_gather(B, N)

    flow_view = (
        flow.reshape(B, 2, H // 8, 8, W // LANE_TILE, LANE_TILE)
        .transpose(0, 1, 2, 4, 3, 5)
        .reshape(-1)
    )

    y1d = event_list[:, :, 1].reshape(-1)
    x1d = event_list[:, :, 2].reshape(-1)

    out1d = gk(flow_view, y1d, x1d)

    out = (
        out1d.reshape(B, ntiles_b, 2, LANE_TILE)
        .transpose(0, 1, 3, 2)
        .reshape(B, n_pad, 2)[:, :N, :]
    )
    return out


# speedup vs baseline: 1.1082x; 1.1082x over previous
import dataclasses
import functools

import jax
import jax.numpy as jnp
from jax import lax
from jax.experimental import pallas as pl
from jax.experimental.pallas import tpu as pltpu
from jax.experimental.pallas import tpu_sc as plsc

H, W = 480, 640
PLANE = H * W
LANES = 16
LANE_TILE = 128
WTILES = W // LANE_TILE
NUM_WORKERS = 32
NUM_CORES = 2


def _build_gather(B, N):
    ntiles_b = (N + LANE_TILE - 1) // LANE_TILE
    t_total = B * ntiles_b
    e_total = B * N
    n_pad = ntiles_b * LANE_TILE
    slots = t_total * 2 * LANE_TILE
    pad_gap = n_pad - N

    tq, rem = divmod(t_total, NUM_WORKERS)
    CT = 49
    NCH = -(-(tq + 1) // CT)
    SRC_WORDS = CT * LANE_TILE
    OUT_WORDS = CT * 2 * LANE_TILE

    mesh = plsc.VectorSubcoreMesh(core_axis_name="c", subcore_axis_name="s")
    cp = pltpu.CompilerParams()
    if "needs_layout_passes" in pltpu.CompilerParams.__dataclass_fields__:
        cp = dataclasses.replace(cp, needs_layout_passes=False)

    @functools.partial(
        pl.kernel,
        out_type=jax.ShapeDtypeStruct((slots,), jnp.float32),
        mesh=mesh,
        compiler_params=cp,
        scratch_types=[
            pltpu.VMEM((SRC_WORDS + pad_gap,), jnp.int32),
            pltpu.VMEM((SRC_WORDS + pad_gap,), jnp.int32),
            pltpu.VMEM((SRC_WORDS + pad_gap,), jnp.int32),
            pltpu.VMEM((SRC_WORDS + pad_gap,), jnp.int32),
            pltpu.VMEM((OUT_WORDS,), jnp.int32),
            pltpu.VMEM((OUT_WORDS,), jnp.int32),
            pltpu.VMEM((OUT_WORDS,), jnp.float32),
            pltpu.VMEM((OUT_WORDS,), jnp.float32),
            pltpu.SemaphoreType.DMA,
            pltpu.SemaphoreType.DMA,
            pltpu.SemaphoreType.DMA,
            pltpu.SemaphoreType.DMA,
            pltpu.SemaphoreType.DMA,
            pltpu.SemaphoreType.DMA,
        ],
    )
    def gather_kernel(flow_hbm, y_hbm, x_hbm, out_hbm,
                      ybuf0, ybuf1, xbuf0, xbuf1, idx0, idx1, ob0, ob1,
                      isem0, isem1, gsem0, gsem1, osem0, osem1):
        ybufs, xbufs = (ybuf0, ybuf1), (xbuf0, xbuf1)
        idxs, obufs = (idx0, idx1), (ob0, ob1)
        isems, gsems, osems = (isem0, isem1), (gsem0, gsem1), (osem0, osem1)

        cid = lax.axis_index("c")
        sid = lax.axis_index("s")
        wid = sid * NUM_CORES + cid
        t0 = wid * tq + jnp.minimum(wid, rem)
        t1 = t0 + tq + jnp.where(wid < rem, 1, 0)

        zv = jnp.zeros((LANES,), jnp.int32)

        @pl.loop(0, pad_gap, step=LANES)
        def _z(j):
            for s in range(2):
                ybufs[s][pl.ds(SRC_WORDS + j, LANES)] = zv
                xbufs[s][pl.ds(SRC_WORDS + j, LANES)] = zv

        def chunk_geom(i):
            S = jnp.minimum(t0 + i * CT, t1 - CT)
            bS = S // ntiles_b
            eS = S * LANE_TILE - pad_gap * bS
            dma_base = jnp.minimum(eS, e_total - SRC_WORDS)
            return S, bS, eS - dma_base, dma_base

        def start_in(i):
            s = i % 2
            _, _, _, dma_base = chunk_geom(i)
            hy = pltpu.async_copy(
                y_hbm.at[pl.ds(dma_base, SRC_WORDS)],
                ybufs[s].at[pl.ds(0, SRC_WORDS)], isems[s])
            hx = pltpu.async_copy(
                x_hbm.at[pl.ds(dma_base, SRC_WORDS)],
                xbufs[s].at[pl.ds(0, SRC_WORDS)], isems[s])
            return hy, hx

        def compute(i):
            s = i % 2
            S, bS, comp, _ = chunk_geom(i)
            ybuf, xbuf, idxbuf = ybufs[s], xbufs[s], idxs[s]

            @pl.loop(0, CT)
            def _tile(t):
                tt = S + t
                b = tt // ntiles_b
                srcoff = t * LANE_TILE - pad_gap * (b - bS) + comp
                toff = t * (2 * LANE_TILE)
                plane0 = b * (2 * PLANE)

                @pl.loop(0, LANE_TILE, step=LANES)
                def _vec(j):
                    yv = ybuf[pl.ds(srcoff + j, LANES)]
                    xv = xbuf[pl.ds(srcoff + j, LANES)]
                    pidx = (
                        ((yv >> 3) * WTILES + (xv >> 7)) * 1024
                        + ((yv & 7) << 7)
                        + (xv & 127)
                    )
                    idxbuf[pl.ds(toff + j, LANES)] = pidx + (plane0 + PLANE)
                    idxbuf[pl.ds(toff + LANE_TILE + j, LANES)] = pidx + plane0

        def start_gather(i):
            s = i % 2
            return pltpu.async_copy(flow_hbm.at[idxs[s]], obufs[s], gsems[s])

        def start_out(i):
            s = i % 2
            S, _, _, _ = chunk_geom(i)
            return pltpu.async_copy(
                obufs[s],
                out_hbm.at[pl.ds(S * (2 * LANE_TILE), OUT_WORDS)], osems[s])

        h_in = {0: start_in(0), 1: start_in(1)}
        h_g, h_out = {}, {}
        for i in range(NCH):
            hy, hx = h_in[i]
            hy.wait()
            hx.wait()
            compute(i)
            if i >= 1:
                h_g[i - 1].wait()
                h_out[i - 1] = start_out(i - 1)
            if i >= 2:
                h_out[i - 2].wait()
            h_g[i] = start_gather(i)
            if i + 2 < NCH:
                h_in[i + 2] = start_in(i + 2)
        h_g[NCH - 1].wait()
        h_out[NCH - 1] = start_out(NCH - 1)
        h_out[NCH - 2].wait()
        h_out[NCH - 1].wait()

    return gather_kernel, ntiles_b, n_pad, slots


def kernel(flow, event_list, pol_mask, event_mask):
    B, _, h, w = flow.shape
    N = event_list.shape[1]
    gk, ntiles_b, n_pad, slots = _build_gather(B, N)

    flow_view = (
        flow.reshape(B, 2, H // 8, 8, W // LANE_TILE, LANE_TILE)
        .transpose(0, 1, 2, 4, 3, 5)
        .reshape(-1)
    )

    y1d = event_list[:, :, 1].reshape(-1)
    x1d = event_list[:, :, 2].reshape(-1)

    out1d = gk(flow_view, y1d, x1d)

    out = (
        out1d.reshape(B, ntiles_b, 2, LANE_TILE)
        .transpose(0, 1, 3, 2)
        .reshape(B, n_pad, 2)[:, :N, :]
    )
    return out


# speedup vs baseline: 1.4145x; 1.2763x over previous
import dataclasses
import functools

import jax
import jax.numpy as jnp
from jax import lax
from jax.experimental import pallas as pl
from jax.experimental.pallas import tpu as pltpu
from jax.experimental.pallas import tpu_sc as plsc

H, W = 480, 640
PLANE = H * W
LANES = 16
LANE_TILE = 128
WTILES = W // LANE_TILE
NUM_WORKERS = 32
NUM_CORES = 2


def _build_gather(B, N):
    ntiles_b = (N + LANE_TILE - 1) // LANE_TILE
    n_pad = ntiles_b * LANE_TILE
    slots = B * ntiles_b * 2 * LANE_TILE
    wpb = NUM_WORKERS // B

    tq, rem = divmod(ntiles_b, wpb)
    CT = 49
    NCH = -(-(tq + 1) // CT)
    OUT_WORDS = CT * 2 * LANE_TILE

    mesh = plsc.VectorSubcoreMesh(core_axis_name="c", subcore_axis_name="s")
    cp = pltpu.CompilerParams()
    if "needs_layout_passes" in pltpu.CompilerParams.__dataclass_fields__:
        cp = dataclasses.replace(cp, needs_layout_passes=False)

    @functools.partial(
        pl.kernel,
        out_type=jax.ShapeDtypeStruct((slots,), jnp.float32),
        mesh=mesh,
        compiler_params=cp,
        scratch_types=[
            pltpu.VMEM((CT, 2, LANE_TILE), jnp.int32),
            pltpu.VMEM((CT, 2, LANE_TILE), jnp.int32),
            pltpu.VMEM((OUT_WORDS,), jnp.int32),
            pltpu.VMEM((OUT_WORDS,), jnp.int32),
            pltpu.VMEM((OUT_WORDS,), jnp.float32),
            pltpu.VMEM((OUT_WORDS,), jnp.float32),
            pltpu.SemaphoreType.DMA,
            pltpu.SemaphoreType.DMA,
            pltpu.SemaphoreType.DMA,
            pltpu.SemaphoreType.DMA,
            pltpu.SemaphoreType.DMA,
            pltpu.SemaphoreType.DMA,
        ],
    )
    def gather_kernel(flow_hbm, ev_hbm, out_hbm,
                      yx0, yx1, idx0, idx1, ob0, ob1,
                      isem0, isem1, gsem0, gsem1, osem0, osem1):
        yxs, idxs, obufs = (yx0, yx1), (idx0, idx1), (ob0, ob1)
        isems, gsems, osems = (isem0, isem1), (gsem0, gsem1), (osem0, osem1)

        cid = lax.axis_index("c")
        sid = lax.axis_index("s")
        wid = sid * NUM_CORES + cid
        b = wid // wpb
        j = wid % wpb
        nt0 = j * tq + jnp.minimum(j, rem)
        nt1 = nt0 + tq + jnp.where(j < rem, 1, 0)
        plane0 = b * (2 * PLANE)
        obase = b * (ntiles_b * 2 * LANE_TILE)

        def chunk_start(i):
            return jnp.minimum(nt0 + i * CT, nt1 - CT)

        def start_in(i):
            s = i % 2
            ntS = chunk_start(i)
            return pltpu.async_copy(
                ev_hbm.at[b, pl.ds(ntS, CT), pl.ds(1, 2), :],
                yxs[s], isems[s])

        def compute(i):
            s = i % 2
            yx, idxbuf = yxs[s], idxs[s]

            @pl.loop(0, CT)
            def _tile(t):
                toff = t * (2 * LANE_TILE)

                @pl.loop(0, LANE_TILE, step=LANES)
                def _vec(j2):
                    yv = yx[t, 0, pl.ds(j2, LANES)]
                    xv = yx[t, 1, pl.ds(j2, LANES)]
                    pidx = (
                        ((yv >> 3) * WTILES + (xv >> 7)) * 1024
                        + ((yv & 7) << 7)
                        + (xv & 127)
                    )
                    idxbuf[pl.ds(toff + j2, LANES)] = pidx + (plane0 + PLANE)
                    idxbuf[pl.ds(toff + LANE_TILE + j2, LANES)] = pidx + plane0

        def start_gather(i):
            s = i % 2
            return pltpu.async_copy(flow_hbm.at[idxs[s]], obufs[s], gsems[s])

        def start_out(i):
            s = i % 2
            ntS = chunk_start(i)
            return pltpu.async_copy(
                obufs[s],
                out_hbm.at[pl.ds(obase + ntS * (2 * LANE_TILE), OUT_WORDS)],
                osems[s])

        h_in = {0: start_in(0), 1: start_in(1)}
        h_g, h_out = {}, {}
        for i in range(NCH):
            h_in[i].wait()
            compute(i)
            if i >= 1:
                h_g[i - 1].wait()
                h_out[i - 1] = start_out(i - 1)
            if i >= 2:
                h_out[i - 2].wait()
            h_g[i] = start_gather(i)
            if i + 2 < NCH:
                h_in[i + 2] = start_in(i + 2)
        h_g[NCH - 1].wait()
        h_out[NCH - 1] = start_out(NCH - 1)
        h_out[NCH - 2].wait()
        h_out[NCH - 1].wait()

    return gather_kernel, ntiles_b, n_pad, slots


def kernel(flow, event_list, pol_mask, event_mask):
    B, _, h, w = flow.shape
    N = event_list.shape[1]
    gk, ntiles_b, n_pad, slots = _build_gather(B, N)

    flow_view = (
        flow.reshape(B, 2, H // 8, 8, W // LANE_TILE, LANE_TILE)
        .transpose(0, 1, 2, 4, 3, 5)
        .reshape(-1)
    )

    evp = jnp.pad(event_list, ((0, 0), (0, n_pad - N), (0, 0)))
    ev_native = evp.reshape(B, ntiles_b, LANE_TILE, 4).transpose(0, 1, 3, 2)

    out1d = gk(flow_view, ev_native)

    out = (
        out1d.reshape(B, ntiles_b, 2, LANE_TILE)
        .transpose(0, 1, 3, 2)
        .reshape(B, n_pad, 2)[:, :N, :]
    )
    return out


# speedup vs baseline: 1.4177x; 1.0023x over previous
import dataclasses
import functools

import jax
import jax.numpy as jnp
from jax import lax
from jax.experimental import pallas as pl
from jax.experimental.pallas import tpu as pltpu
from jax.experimental.pallas import tpu_sc as plsc

H, W = 480, 640
PLANE = H * W
LANES = 16
LANE_TILE = 128
WTILES = W // LANE_TILE
NUM_WORKERS = 32
NUM_CORES = 2


def _build_gather(B, N):
    ntiles_b = (N + LANE_TILE - 1) // LANE_TILE
    n_pad = ntiles_b * LANE_TILE
    slots = B * ntiles_b * 2 * LANE_TILE
    wpb = NUM_WORKERS // B

    tq, rem = divmod(ntiles_b, wpb)
    CT = 28
    NCH = -(-(tq + 1) // CT)
    OUT_WORDS = CT * 2 * LANE_TILE

    mesh = plsc.VectorSubcoreMesh(core_axis_name="c", subcore_axis_name="s")
    cp = pltpu.CompilerParams()
    if "needs_layout_passes" in pltpu.CompilerParams.__dataclass_fields__:
        cp = dataclasses.replace(cp, needs_layout_passes=False)

    @functools.partial(
        pl.kernel,
        out_type=jax.ShapeDtypeStruct((slots,), jnp.float32),
        mesh=mesh,
        compiler_params=cp,
        scratch_types=[
            pltpu.VMEM((CT, 2, LANE_TILE), jnp.int32),
            pltpu.VMEM((CT, 2, LANE_TILE), jnp.int32),
            pltpu.VMEM((OUT_WORDS,), jnp.int32),
            pltpu.VMEM((OUT_WORDS,), jnp.int32),
            pltpu.VMEM((OUT_WORDS,), jnp.float32),
            pltpu.VMEM((OUT_WORDS,), jnp.float32),
            pltpu.SemaphoreType.DMA,
            pltpu.SemaphoreType.DMA,
            pltpu.SemaphoreType.DMA,
            pltpu.SemaphoreType.DMA,
            pltpu.SemaphoreType.DMA,
            pltpu.SemaphoreType.DMA,
        ],
    )
    def gather_kernel(flow_hbm, ev_hbm, out_hbm,
                      yx0, yx1, idx0, idx1, ob0, ob1,
                      isem0, isem1, gsem0, gsem1, osem0, osem1):
        yxs, idxs, obufs = (yx0, yx1), (idx0, idx1), (ob0, ob1)
        isems, gsems, osems = (isem0, isem1), (gsem0, gsem1), (osem0, osem1)

        cid = lax.axis_index("c")
        sid = lax.axis_index("s")
        wid = sid * NUM_CORES + cid
        b = wid // wpb
        j = wid % wpb
        nt0 = j * tq + jnp.minimum(j, rem)
        nt1 = nt0 + tq + jnp.where(j < rem, 1, 0)
        plane0 = b * (2 * PLANE)
        obase = b * (ntiles_b * 2 * LANE_TILE)

        def chunk_start(i):
            return jnp.minimum(nt0 + i * CT, nt1 - CT)

        def start_in(i):
            s = i % 2
            ntS = chunk_start(i)
            return pltpu.async_copy(
                ev_hbm.at[b, pl.ds(ntS, CT), pl.ds(1, 2), :],
                yxs[s], isems[s])

        def compute(i):
            s = i % 2
            yx, idxbuf = yxs[s], idxs[s]

            @pl.loop(0, CT)
            def _tile(t):
                toff = t * (2 * LANE_TILE)

                @pl.loop(0, LANE_TILE, step=LANES)
                def _vec(j2):
                    yv = yx[t, 0, pl.ds(j2, LANES)]
                    xv = yx[t, 1, pl.ds(j2, LANES)]
                    pidx = (
                        ((yv >> 3) * WTILES + (xv >> 7)) * 1024
                        + ((yv & 7) << 7)
                        + (xv & 127)
                    )
                    idxbuf[pl.ds(toff + j2, LANES)] = pidx + (plane0 + PLANE)
                    idxbuf[pl.ds(toff + LANE_TILE + j2, LANES)] = pidx + plane0

        def start_gather(i):
            s = i % 2
            return pltpu.async_copy(flow_hbm.at[idxs[s]], obufs[s], gsems[s])

        def start_out(i):
            s = i % 2
            ntS = chunk_start(i)
            return pltpu.async_copy(
                obufs[s],
                out_hbm.at[pl.ds(obase + ntS * (2 * LANE_TILE), OUT_WORDS)],
                osems[s])

        h_in = {0: start_in(0), 1: start_in(1)}
        h_g, h_out = {}, {}
        for i in range(NCH):
            h_in[i].wait()
            compute(i)
            if i >= 1:
                h_g[i - 1].wait()
                h_out[i - 1] = start_out(i - 1)
            if i >= 2:
                h_out[i - 2].wait()
            h_g[i] = start_gather(i)
            if i + 2 < NCH:
                h_in[i + 2] = start_in(i + 2)
        h_g[NCH - 1].wait()
        h_out[NCH - 1] = start_out(NCH - 1)
        h_out[NCH - 2].wait()
        h_out[NCH - 1].wait()

    return gather_kernel, ntiles_b, n_pad, slots


def kernel(flow, event_list, pol_mask, event_mask):
    B, _, h, w = flow.shape
    N = event_list.shape[1]
    gk, ntiles_b, n_pad, slots = _build_gather(B, N)

    flow_view = (
        flow.reshape(B, 2, H // 8, 8, W // LANE_TILE, LANE_TILE)
        .transpose(0, 1, 2, 4, 3, 5)
        .reshape(-1)
    )

    evp = jnp.pad(event_list, ((0, 0), (0, n_pad - N), (0, 0)))
    ev_native = evp.reshape(B, ntiles_b, LANE_TILE, 4).transpose(0, 1, 3, 2)

    out1d = gk(flow_view, ev_native)

    out = (
        out1d.reshape(B, ntiles_b, 2, LANE_TILE)
        .transpose(0, 1, 3, 2)
        .reshape(B, n_pad, 2)[:, :N, :]
    )
    return out


# speedup vs baseline: 1.4291x; 1.0080x over previous
import dataclasses
import functools

import jax
import jax.numpy as jnp
from jax import lax
from jax.experimental import pallas as pl
from jax.experimental.pallas import tpu as pltpu
from jax.experimental.pallas import tpu_sc as plsc

H, W = 480, 640
PLANE = H * W
LANES = 16
LANE_TILE = 128
WTILES = W // LANE_TILE
NUM_WORKERS = 32
NUM_CORES = 2


def _build_gather(B, N):
    ntiles_b = (N + LANE_TILE - 1) // LANE_TILE
    n_pad = ntiles_b * LANE_TILE
    slots = B * ntiles_b * 2 * LANE_TILE
    wpb = NUM_WORKERS // B

    tq, rem = divmod(ntiles_b, wpb)
    CT = 28
    NCH = -(-(tq + 1) // CT)
    OUT_WORDS = CT * 2 * LANE_TILE

    mesh = plsc.VectorSubcoreMesh(core_axis_name="c", subcore_axis_name="s")
    cp = pltpu.CompilerParams()
    if "needs_layout_passes" in pltpu.CompilerParams.__dataclass_fields__:
        cp = dataclasses.replace(cp, needs_layout_passes=False)

    @functools.partial(
        pl.kernel,
        out_type=jax.ShapeDtypeStruct((slots,), jnp.float32),
        mesh=mesh,
        compiler_params=cp,
        scratch_types=[
            pltpu.VMEM((CT, 2, LANE_TILE), jnp.int32),
            pltpu.VMEM((CT, 2, LANE_TILE), jnp.int32),
            pltpu.VMEM((OUT_WORDS,), jnp.int32),
            pltpu.VMEM((OUT_WORDS,), jnp.int32),
            pltpu.VMEM((OUT_WORDS,), jnp.float32),
            pltpu.VMEM((OUT_WORDS,), jnp.float32),
            pltpu.SemaphoreType.DMA,
            pltpu.SemaphoreType.DMA,
            pltpu.SemaphoreType.DMA,
            pltpu.SemaphoreType.DMA,
            pltpu.SemaphoreType.DMA,
            pltpu.SemaphoreType.DMA,
        ],
    )
    def gather_kernel(flow_hbm, ev_hbm, out_hbm,
                      yx0, yx1, idx0, idx1, ob0, ob1,
                      isem0, isem1, gsem0, gsem1, osem0, osem1):
        yxs, idxs, obufs = (yx0, yx1), (idx0, idx1), (ob0, ob1)
        isems, gsems, osems = (isem0, isem1), (gsem0, gsem1), (osem0, osem1)

        cid = lax.axis_index("c")
        sid = lax.axis_index("s")
        wid = sid * NUM_CORES + cid
        b = wid // wpb
        j = wid % wpb
        nt0 = j * tq + jnp.minimum(j, rem)
        nt1 = nt0 + tq + jnp.where(j < rem, 1, 0)
        plane0 = b * (2 * PLANE)
        obase = b * (ntiles_b * 2 * LANE_TILE)

        def chunk_start(i):
            return jnp.minimum(nt0 + i * CT, nt1 - CT)

        def start_in(i):
            s = i % 2
            ntS = chunk_start(i)
            return pltpu.async_copy(
                ev_hbm.at[b, pl.ds(ntS, CT), pl.ds(1, 2), :],
                yxs[s], isems[s])

        def compute(i):
            s = i % 2
            yx, idxbuf = yxs[s], idxs[s]

            @pl.loop(0, CT)
            def _tile(t):
                toff = t * (2 * LANE_TILE)

                @pl.loop(0, LANE_TILE, step=LANES)
                def _vec(j2):
                    yv = yx[t, 0, pl.ds(j2, LANES)]
                    xv = yx[t, 1, pl.ds(j2, LANES)]
                    pidx = (
                        ((yv >> 3) * WTILES + (xv >> 7)) * 1024
                        + ((yv & 7) << 7)
                        + (xv & 127)
                    )
                    idxbuf[pl.ds(toff + j2, LANES)] = pidx + (plane0 + PLANE)
                    idxbuf[pl.ds(toff + LANE_TILE + j2, LANES)] = pidx + plane0

        def start_gather(i):
            s = i % 2
            return pltpu.async_copy(flow_hbm.at[idxs[s]], obufs[s], gsems[s])

        def start_out(i):
            s = i % 2
            ntS = chunk_start(i)
            return pltpu.async_copy(
                obufs[s],
                out_hbm.at[pl.ds(obase + ntS * (2 * LANE_TILE), OUT_WORDS)],
                osems[s])

        h_in = {0: start_in(0), 1: start_in(1)}
        h_g, h_out = {}, {}
        for i in range(NCH):
            h_in[i].wait()
            compute(i)
            if i >= 2:
                h_out[i - 2].wait()
            h_g[i] = start_gather(i)
            if i >= 1:
                h_g[i - 1].wait()
                h_out[i - 1] = start_out(i - 1)
            if i + 2 < NCH:
                h_in[i + 2] = start_in(i + 2)
        h_g[NCH - 1].wait()
        h_out[NCH - 1] = start_out(NCH - 1)
        h_out[NCH - 2].wait()
        h_out[NCH - 1].wait()

    return gather_kernel, ntiles_b, n_pad, slots


def kernel(flow, event_list, pol_mask, event_mask):
    B, _, h, w = flow.shape
    N = event_list.shape[1]
    gk, ntiles_b, n_pad, slots = _build_gather(B, N)

    flow_view = (
        flow.reshape(B, 2, H // 8, 8, W // LANE_TILE, LANE_TILE)
        .transpose(0, 1, 2, 4, 3, 5)
        .reshape(-1)
    )

    evp = jnp.pad(event_list, ((0, 0), (0, n_pad - N), (0, 0)))
    ev_native = evp.reshape(B, ntiles_b, LANE_TILE, 4).transpose(0, 1, 3, 2)

    out1d = gk(flow_view, ev_native)

    out = (
        out1d.reshape(B, ntiles_b, 2, LANE_TILE)
        .transpose(0, 1, 3, 2)
        .reshape(B, n_pad, 2)[:, :N, :]
    )
    return out
